# baseline (speedup 1.0000x reference)
import functools

import jax
import jax.numpy as jnp
from jax import lax
from jax.experimental import pallas as pl
from jax.experimental.pallas import tpu as pltpu
from jax.experimental.pallas import tpu_sc as plsc

VOCAB = 100000
DIM = 1024
BATCH = 4
SEQ = 2048

_info = plsc.get_sparse_core_info()
NC, NS = _info.num_cores, _info.num_subcores
NW = NC * NS
TOTAL_ROWS = BATCH * SEQ
ROWS_PER_W = TOTAL_ROWS // NW
SUB = 16
GROUP = 32
SUBS_PER_GROUP = GROUP // SUB
N_GROUPS = ROWS_PER_W // GROUP
NBUF = 3


def _gather_body(idx_hbm, table_hbm, out_hbm, idx_v, *rest):
    bufs = rest[:NBUF]
    gsems = rest[NBUF:2 * NBUF]
    osems = rest[2 * NBUF:3 * NBUF]
    wid = lax.axis_index("s") * NC + lax.axis_index("c")
    base = wid * ROWS_PER_W
    per_batch = SEQ // ROWS_PER_W
    b = wid // per_batch
    off = (wid % per_batch) * ROWS_PER_W
    pltpu.sync_copy(idx_hbm.at[b, pl.ds(off, ROWS_PER_W)], idx_v)

    def fill(g):
        buf = bufs[g % NBUF]
        sem = gsems[g % NBUF]
        return [
            pltpu.async_copy(
                table_hbm.at[idx_v.at[pl.ds(g * GROUP + s * SUB, SUB)]],
                buf.at[pl.ds(s * SUB, SUB)], sem)
            for s in range(SUBS_PER_GROUP)
        ]

    pend_g = [None] * N_GROUPS
    pend_o = [None] * N_GROUPS
    for g in range(min(NBUF, N_GROUPS)):
        pend_g[g] = fill(g)
    for c in range(N_GROUPS):
        for h in pend_g[c]:
            h.wait()
        pend_o[c] = pltpu.async_copy(
            bufs[c % NBUF], out_hbm.at[pl.ds(base + c * GROUP, GROUP)],
            osems[c % NBUF])
        g = c + NBUF - 1
        if c >= 1 and g < N_GROUPS:
            pend_o[c - 1].wait()
            pend_o[c - 1] = None
            pend_g[g] = fill(g)
    for o in pend_o:
        if o is not None:
            o.wait()


@jax.jit
def _embedding_gather(ids, table):
    mesh = plsc.VectorSubcoreMesh(core_axis_name="c", subcore_axis_name="s")
    kern = functools.partial(
        pl.kernel,
        mesh=mesh,
        out_type=jax.ShapeDtypeStruct((TOTAL_ROWS, DIM), jnp.float32),
        scratch_types=(
            [pltpu.VMEM((ROWS_PER_W,), jnp.int32)]
            + [pltpu.VMEM((GROUP, DIM), jnp.float32)] * NBUF
            + [pltpu.SemaphoreType.DMA] * (2 * NBUF)
        ),
    )(_gather_body)
    return kern(ids, table)


def kernel(input_ids, embed_tokens, cos_cached, sin_cached):
    seq_len = input_ids.shape[1]
    rows = _embedding_gather(input_ids, embed_tokens)
    x = rows.reshape(input_ids.shape[0], seq_len, DIM)
    cos = cos_cached[:seq_len].astype(x.dtype)
    sin = sin_cached[:seq_len].astype(x.dtype)
    return (x, cos, sin)


# speedup vs baseline: 1.0073x; 1.0073x over previous
import functools

import jax
import jax.numpy as jnp
from jax import lax
from jax.experimental import pallas as pl
from jax.experimental.pallas import tpu as pltpu
from jax.experimental.pallas import tpu_sc as plsc

VOCAB = 100000
DIM = 1024
BATCH = 4
SEQ = 2048

_info = plsc.get_sparse_core_info()
NC, NS = _info.num_cores, _info.num_subcores
NW = NC * NS
TOTAL_ROWS = BATCH * SEQ
ROWS_PER_W = TOTAL_ROWS // NW
CHUNK = 16
N_CHUNKS = ROWS_PER_W // CHUNK
NBUF = 6


def _gather_body(idx_hbm, table_hbm, out_hbm, idx_v, *rest):
    bufs = rest[:NBUF]
    gsems = rest[NBUF:2 * NBUF]
    osems = rest[2 * NBUF:3 * NBUF]
    wid = lax.axis_index("s") * NC + lax.axis_index("c")
    base = wid * ROWS_PER_W
    per_batch = SEQ // ROWS_PER_W
    b = wid // per_batch
    off = (wid % per_batch) * ROWS_PER_W
    pltpu.sync_copy(idx_hbm.at[b, pl.ds(off, ROWS_PER_W)], idx_v)

    pend_g = [None] * N_CHUNKS
    pend_o = [None] * N_CHUNKS
    for c in range(min(NBUF, N_CHUNKS)):
        pend_g[c] = pltpu.async_copy(
            table_hbm.at[idx_v.at[pl.ds(c * CHUNK, CHUNK)]], bufs[c],
            gsems[c])
    for c in range(N_CHUNKS):
        pend_g[c].wait()
        pend_o[c] = pltpu.async_copy(
            bufs[c % NBUF], out_hbm.at[pl.ds(base + c * CHUNK, CHUNK)],
            osems[c % NBUF])
        g = c + NBUF - 1
        if c >= 1 and g < N_CHUNKS:
            pend_o[c - 1].wait()
            pend_o[c - 1] = None
            pend_g[g] = pltpu.async_copy(
                table_hbm.at[idx_v.at[pl.ds(g * CHUNK, CHUNK)]],
                bufs[g % NBUF], gsems[g % NBUF])
    for o in pend_o:
        if o is not None:
            o.wait()


@jax.jit
def _embedding_gather(ids, table):
    mesh = plsc.VectorSubcoreMesh(core_axis_name="c", subcore_axis_name="s")
    kern = functools.partial(
        pl.kernel,
        mesh=mesh,
        out_type=jax.ShapeDtypeStruct((TOTAL_ROWS, DIM), jnp.float32),
        scratch_types=(
            [pltpu.VMEM((ROWS_PER_W,), jnp.int32)]
            + [pltpu.VMEM((CHUNK, DIM), jnp.float32)] * NBUF
            + [pltpu.SemaphoreType.DMA] * (2 * NBUF)
        ),
    )(_gather_body)
    return kern(ids, table)


def _rope_copy_body(cos_ref, sin_ref, cos_out, sin_out):
    cos_out[...] = cos_ref[...]
    sin_out[...] = sin_ref[...]


@jax.jit
def _rope_copy(cos_c, sin_c):
    return pl.pallas_call(
        _rope_copy_body,
        out_shape=(
            jax.ShapeDtypeStruct(cos_c.shape, cos_c.dtype),
            jax.ShapeDtypeStruct(sin_c.shape, sin_c.dtype),
        ),
    )(cos_c, sin_c)


def kernel(input_ids, embed_tokens, cos_cached, sin_cached):
    seq_len = input_ids.shape[1]
    rows = _embedding_gather(input_ids, embed_tokens)
    x = rows.reshape(input_ids.shape[0], seq_len, DIM)
    cos, sin = _rope_copy(cos_cached[:seq_len].astype(x.dtype),
                          sin_cached[:seq_len].astype(x.dtype))
    return (x, cos, sin)


# speedup vs baseline: 1.0358x; 1.0283x over previous
import functools

import jax
import jax.numpy as jnp
from jax import lax
from jax.experimental import pallas as pl
from jax.experimental.pallas import tpu as pltpu
from jax.experimental.pallas import tpu_sc as plsc

VOCAB = 100000
DIM = 1024
BATCH = 4
SEQ = 2048

_info = plsc.get_sparse_core_info()
NC, NS = _info.num_cores, _info.num_subcores
NW = NC * NS
TOTAL_ROWS = BATCH * SEQ
ROWS_PER_W = TOTAL_ROWS // NW
CHUNK = 16
N_CHUNKS = ROWS_PER_W // CHUNK
NBUF = 7


def _gather_body(idx_hbm, table_hbm, out_hbm, idx_v, *rest):
    bufs = rest[:NBUF]
    gsems = rest[NBUF:2 * NBUF]
    osems = rest[2 * NBUF:3 * NBUF]
    wid = lax.axis_index("s") * NC + lax.axis_index("c")
    base = wid * ROWS_PER_W
    per_batch = SEQ // ROWS_PER_W
    b = wid // per_batch
    off = (wid % per_batch) * ROWS_PER_W
    pltpu.sync_copy(idx_hbm.at[b, pl.ds(off, ROWS_PER_W)], idx_v)

    pend_g = [None] * N_CHUNKS
    pend_o = [None] * N_CHUNKS
    for c in range(min(NBUF, N_CHUNKS)):
        pend_g[c] = pltpu.async_copy(
            table_hbm.at[idx_v.at[pl.ds(c * CHUNK, CHUNK)]], bufs[c],
            gsems[c])
    for c in range(N_CHUNKS):
        pend_g[c].wait()
        pend_o[c] = pltpu.async_copy(
            bufs[c % NBUF], out_hbm.at[pl.ds(base + c * CHUNK, CHUNK)],
            osems[c % NBUF])
        g = c + NBUF - 1
        if c >= 1 and g < N_CHUNKS:
            pend_o[c - 1].wait()
            pend_o[c - 1] = None
            pend_g[g] = pltpu.async_copy(
                table_hbm.at[idx_v.at[pl.ds(g * CHUNK, CHUNK)]],
                bufs[g % NBUF], gsems[g % NBUF])
    for o in pend_o:
        if o is not None:
            o.wait()


@jax.jit
def _embedding_gather(ids, table):
    mesh = plsc.VectorSubcoreMesh(core_axis_name="c", subcore_axis_name="s")
    kern = functools.partial(
        pl.kernel,
        mesh=mesh,
        out_type=jax.ShapeDtypeStruct((TOTAL_ROWS, DIM), jnp.float32),
        scratch_types=(
            [pltpu.VMEM((ROWS_PER_W,), jnp.int32)]
            + [pltpu.VMEM((CHUNK, DIM), jnp.float32)] * NBUF
            + [pltpu.SemaphoreType.DMA] * (2 * NBUF)
        ),
    )(_gather_body)
    return kern(ids, table)


def kernel(input_ids, embed_tokens, cos_cached, sin_cached):
    seq_len = input_ids.shape[1]
    rows = _embedding_gather(input_ids, embed_tokens)
    x = rows.reshape(input_ids.shape[0], seq_len, DIM)
    cos = cos_cached[:seq_len].astype(x.dtype)
    sin = sin_cached[:seq_len].astype(x.dtype)
    return (x, cos, sin)


# speedup vs baseline: 1.0419x; 1.0059x over previous
import functools

import jax
import jax.numpy as jnp
from jax import lax
from jax.experimental import pallas as pl
from jax.experimental.pallas import tpu as pltpu
from jax.experimental.pallas import tpu_sc as plsc

VOCAB = 100000
DIM = 1024
BATCH = 4
SEQ = 2048

_info = plsc.get_sparse_core_info()
NC, NS = _info.num_cores, _info.num_subcores
NW = NC * NS
TOTAL_ROWS = BATCH * SEQ
ROWS_PER_W = TOTAL_ROWS // NW
CHUNK = 16
N_CHUNKS = ROWS_PER_W // CHUNK
NBUF = 6


def _gather_body(idx_hbm, table_hbm, out_hbm, idx_v, *rest):
    bufs = rest[:NBUF]
    gsems = rest[NBUF:2 * NBUF]
    osems = rest[2 * NBUF:3 * NBUF]
    wid = lax.axis_index("s") * NC + lax.axis_index("c")
    base = wid * ROWS_PER_W
    per_batch = SEQ // ROWS_PER_W
    b = wid // per_batch
    off = (wid % per_batch) * ROWS_PER_W
    pltpu.sync_copy(idx_hbm.at[b, pl.ds(off, ROWS_PER_W)], idx_v)

    pend_g = [None] * N_CHUNKS
    pend_o = [None] * N_CHUNKS
    for c in range(min(NBUF, N_CHUNKS)):
        pend_g[c] = pltpu.async_copy(
            table_hbm.at[idx_v.at[pl.ds(c * CHUNK, CHUNK)]], bufs[c],
            gsems[c])
    for c in range(N_CHUNKS):
        pend_g[c].wait()
        pend_o[c] = pltpu.async_copy(
            bufs[c % NBUF], out_hbm.at[pl.ds(base + c * CHUNK, CHUNK)],
            osems[c % NBUF])
        g = c + NBUF - 1
        if c >= 1 and g < N_CHUNKS:
            pend_o[c - 1].wait()
            pend_o[c - 1] = None
            pend_g[g] = pltpu.async_copy(
                table_hbm.at[idx_v.at[pl.ds(g * CHUNK, CHUNK)]],
                bufs[g % NBUF], gsems[g % NBUF])
    for o in pend_o:
        if o is not None:
            o.wait()


@jax.jit
def _embedding_gather(ids, table):
    mesh = plsc.VectorSubcoreMesh(core_axis_name="c", subcore_axis_name="s")
    kern = functools.partial(
        pl.kernel,
        mesh=mesh,
        out_type=jax.ShapeDtypeStruct((TOTAL_ROWS, DIM), jnp.float32),
        scratch_types=(
            [pltpu.VMEM((ROWS_PER_W,), jnp.int32)]
            + [pltpu.VMEM((CHUNK, DIM), jnp.float32)] * NBUF
            + [pltpu.SemaphoreType.DMA] * (2 * NBUF)
        ),
    )(_gather_body)
    return kern(ids, table)


def kernel(input_ids, embed_tokens, cos_cached, sin_cached):
    seq_len = input_ids.shape[1]
    rows = _embedding_gather(input_ids, embed_tokens)
    x = rows.reshape(input_ids.shape[0], seq_len, DIM)
    cos = cos_cached[:seq_len].astype(x.dtype)
    sin = sin_cached[:seq_len].astype(x.dtype)
    return (x, cos, sin)
